# baseline (speedup 1.0000x reference)
import functools

import jax
import jax.numpy as jnp
from jax import lax
from jax.experimental import pallas as pl
from jax.experimental.pallas import tpu as pltpu
from jax.experimental.pallas import tpu_sc as plsc

NPOINT = 512
NSAMPLE = 32


def _fps_body(xr_ref, yr_ref, zr_ref, fps_ref, nx_ref, ny_ref, nz_ref,
              dmin_ref):
    X = xr_ref[...]
    Y = yr_ref[...]
    Z = zr_ref[...]
    B = X.shape[0]
    pos = (lax.broadcasted_iota(jnp.int32, X.shape, 1) * 128
           + lax.broadcasted_iota(jnp.int32, X.shape, 2))
    lane = lax.broadcasted_iota(jnp.int32, (B, NPOINT), 1)
    BIGI = jnp.int32(1 << 30)

    def coords_at(sel):
        m = pos == sel[:, None, None]
        cx = jnp.sum(jnp.where(m, X, 0.0), axis=(1, 2))
        cy = jnp.sum(jnp.where(m, Y, 0.0), axis=(1, 2))
        cz = jnp.sum(jnp.where(m, Z, 0.0), axis=(1, 2))
        return cx, cy, cz

    cx, cy, cz = coords_at(jnp.zeros((B,), jnp.int32))
    fps_ref[...] = jnp.zeros((B, NPOINT), jnp.int32)
    nx_ref[...] = jnp.where(lane == 0, cx[:, None], 0.0)
    ny_ref[...] = jnp.where(lane == 0, cy[:, None], 0.0)
    nz_ref[...] = jnp.where(lane == 0, cz[:, None], 0.0)
    dmin_ref[...] = jnp.full(X.shape, jnp.inf, jnp.float32)

    def body(i, carry):
        cx, cy, cz = carry
        d = ((X - cx[:, None, None]) ** 2 + (Y - cy[:, None, None]) ** 2
             + (Z - cz[:, None, None]) ** 2)
        dmin = jnp.minimum(dmin_ref[...], d)
        dmin_ref[...] = dmin
        m = jnp.max(dmin, axis=(1, 2))
        sel = jnp.min(jnp.where(dmin == m[:, None, None], pos, BIGI),
                      axis=(1, 2))
        fps_ref[...] = jnp.where(lane == i, sel[:, None], fps_ref[...])
        cx, cy, cz = coords_at(sel)
        nx_ref[...] = jnp.where(lane == i, cx[:, None], nx_ref[...])
        ny_ref[...] = jnp.where(lane == i, cy[:, None], ny_ref[...])
        nz_ref[...] = jnp.where(lane == i, cz[:, None], nz_ref[...])
        return cx, cy, cz

    lax.fori_loop(1, NPOINT, body, (cx, cy, cz))


def _fps(xr, yr, zr):
    B = xr.shape[0]
    return pl.pallas_call(
        _fps_body,
        out_shape=[jax.ShapeDtypeStruct((B, NPOINT), jnp.int32),
                   jax.ShapeDtypeStruct((B, NPOINT), jnp.float32),
                   jax.ShapeDtypeStruct((B, NPOINT), jnp.float32),
                   jax.ShapeDtypeStruct((B, NPOINT), jnp.float32)],
        scratch_shapes=[pltpu.VMEM(xr.shape, jnp.float32)],
    )(xr, yr, zr)


_SB = 128


def _knn_body(p8_ref, qs_ref, idx_ref, d_ref):
    b = pl.program_id(0)
    N = p8_ref.shape[1]
    P = p8_ref[0]
    QT = qs_ref[0]
    qx = QT[0:1]
    qy = QT[1:2]
    qz = QT[2:3]
    q2 = qx * qx + qy * qy + qz * qz
    E = lax.dot_general(P.astype(jnp.bfloat16), QT.astype(jnp.bfloat16),
                        (((1,), (0,)), ((), ())),
                        preferred_element_type=jnp.float32)
    p2col = P[:, 3:4]
    Dv = (-2.0 * E + q2) + p2col
    u = lax.bitcast_convert_type(Dv, jnp.int32)
    key = u ^ ((u >> 31) & jnp.int32(0x7FFFFFFF))
    d_ref[...] = key

    BIGI = jnp.int32(1 << 30)
    MAXI = jnp.int32(0x7FFFFFFF)

    def body(k, carry):
        mk, ak = carry
        KEY = d_ref[...]
        rid = lax.broadcasted_iota(jnp.int32, KEY.shape, 0)
        gt = KEY > mk
        eq = KEY == mk
        m = jnp.min(jnp.where(gt | (eq & (rid > ak)), KEY, MAXI), axis=0,
                    keepdims=True)
        a = jnp.min(jnp.where((KEY == m) & ((m > mk) | (rid > ak)), rid,
                              BIGI), axis=0, keepdims=True)
        kio = lax.broadcasted_iota(jnp.int32, idx_ref.shape, 2)
        idx_ref[...] = jnp.where(kio == k, (a + b * N)[:, :, None],
                                 idx_ref[...])
        return m, a

    mk0 = jnp.full((1, _SB), jnp.int32(-2147483647 - 1))
    ak0 = jnp.full((1, _SB), jnp.int32(-1))
    lax.fori_loop(0, NSAMPLE, body, (mk0, ak0))


def _knn(p8, qs):
    B, N, _ = p8.shape
    return pl.pallas_call(
        _knn_body,
        grid=(B, NPOINT // _SB),
        in_specs=[
            pl.BlockSpec((1, N, 8), lambda b, s: (b, 0, 0)),
            pl.BlockSpec((1, 8, _SB), lambda b, s: (b, 0, s)),
        ],
        out_specs=pl.BlockSpec((1, _SB, NSAMPLE), lambda b, s: (b, s, 0)),
        out_shape=jax.ShapeDtypeStruct((B, NPOINT, NSAMPLE), jnp.int32),
        scratch_shapes=[pltpu.VMEM((N, _SB), jnp.int32)],
    )(p8, qs)


_NC, _NS = 2, 16
_NW = _NC * _NS
_CH = 128


def _gather_sc(pts, idx2d):
    n_rows = idx2d.shape[0] * idx2d.shape[1]
    rows_w = n_rows // _NW
    nch = rows_w // _CH
    d = pts.shape[1]
    mesh = plsc.VectorSubcoreMesh(core_axis_name="c", subcore_axis_name="s")

    @functools.partial(
        pl.kernel,
        out_type=jax.ShapeDtypeStruct((n_rows, d), jnp.float32),
        mesh=mesh,
        compiler_params=pltpu.CompilerParams(use_tc_tiling_on_sc=False),
        scratch_types=[
            pltpu.VMEM((nch, _CH), jnp.int32),
            pltpu.VMEM((_CH, d), jnp.float32),
            pltpu.VMEM((_CH, d), jnp.float32),
            pltpu.SemaphoreType.DMA,
            pltpu.SemaphoreType.DMA,
        ],
    )
    def k(pts_hbm, idx_hbm, out_hbm, idx_v, buf0, buf1, sem0, sem1):
        wid = lax.axis_index("s") * _NC + lax.axis_index("c")
        pltpu.sync_copy(idx_hbm.at[pl.ds(wid * nch, nch)], idx_v)
        obase = wid * rows_w

        def pair(jj, carry):
            j0 = 2 * jj
            c0 = pltpu.async_copy(pts_hbm.at[idx_v.at[j0]], buf0, sem0)
            c1 = pltpu.async_copy(pts_hbm.at[idx_v.at[j0 + 1]], buf1, sem1)
            c0.wait()
            pltpu.sync_copy(buf0, out_hbm.at[pl.ds(obase + j0 * _CH, _CH)])
            c1.wait()
            pltpu.sync_copy(buf1,
                            out_hbm.at[pl.ds(obase + (j0 + 1) * _CH, _CH)])
            return carry

        lax.fori_loop(0, nch // 2, pair, 0)

    return k(pts, idx2d)


_BLK = 2048


def _stats_x_body(nbr_ref, ce_ref, g_ref, s_ref):
    p = pl.program_id(0)
    nb = nbr_ref[...]
    ce = ce_ref[...]
    x = jnp.concatenate([nb - ce, ce], axis=1)
    g = lax.dot_general(x, x, (((0,), (0,)), ((), ())),
                        preferred_element_type=jnp.float32,
                        precision=lax.Precision.HIGHEST)
    ones = jnp.ones((8, _BLK), jnp.float32)
    s = lax.dot_general(ones, x, (((1,), (0,)), ((), ())),
                        preferred_element_type=jnp.float32,
                        precision=lax.Precision.HIGHEST)

    @pl.when(p == 0)
    def _():
        g_ref[...] = jnp.zeros_like(g_ref)
        s_ref[...] = jnp.zeros_like(s_ref)

    g_ref[...] += g
    s_ref[...] += s


def _stats_x(nbr, ce):
    m = nbr.shape[0]
    return pl.pallas_call(
        _stats_x_body,
        grid=(m // _BLK,),
        in_specs=[pl.BlockSpec((_BLK, 32), lambda p: (p, 0)),
                  pl.BlockSpec((_BLK, 32), lambda p: (p, 0))],
        out_specs=[pl.BlockSpec((64, 64), lambda p: (0, 0)),
                   pl.BlockSpec((8, 64), lambda p: (0, 0))],
        out_shape=[jax.ShapeDtypeStruct((64, 64), jnp.float32),
                   jax.ShapeDtypeStruct((8, 64), jnp.float32)],
    )(nbr, ce)


def _swish_k(t):
    return t * (1.0 / (1.0 + jnp.exp(-t)))


def _out1_of(nb, ce, w1_ref, b1_ref):
    x = jnp.concatenate([nb - ce, ce], axis=1)
    t = lax.dot_general(x, w1_ref[...], (((1,), (1,)), ((), ())),
                        preferred_element_type=jnp.float32) + b1_ref[0:1]
    return x + _swish_k(t)


def _stats_o1_body(nbr_ref, ce_ref, w1_ref, b1_ref, g_ref, s_ref):
    p = pl.program_id(0)
    o1 = _out1_of(nbr_ref[...], ce_ref[...], w1_ref, b1_ref)
    g = lax.dot_general(o1, o1, (((0,), (0,)), ((), ())),
                        preferred_element_type=jnp.float32,
                        precision=lax.Precision.HIGHEST)
    ones = jnp.ones((8, _BLK), jnp.float32)
    s = lax.dot_general(ones, o1, (((1,), (0,)), ((), ())),
                        preferred_element_type=jnp.float32,
                        precision=lax.Precision.HIGHEST)

    @pl.when(p == 0)
    def _():
        g_ref[...] = jnp.zeros_like(g_ref)
        s_ref[...] = jnp.zeros_like(s_ref)

    g_ref[...] += g
    s_ref[...] += s


def _stats_o1(nbr, ce, w1p, b1t):
    m = nbr.shape[0]
    return pl.pallas_call(
        _stats_o1_body,
        grid=(m // _BLK,),
        in_specs=[pl.BlockSpec((_BLK, 32), lambda p: (p, 0)),
                  pl.BlockSpec((_BLK, 32), lambda p: (p, 0)),
                  pl.BlockSpec((64, 64), lambda p: (0, 0)),
                  pl.BlockSpec((8, 64), lambda p: (0, 0))],
        out_specs=[pl.BlockSpec((64, 64), lambda p: (0, 0)),
                   pl.BlockSpec((8, 64), lambda p: (0, 0))],
        out_shape=[jax.ShapeDtypeStruct((64, 64), jnp.float32),
                   jax.ShapeDtypeStruct((8, 64), jnp.float32)],
    )(nbr, ce, w1p, b1t)


def _final_body(nbr_ref, ce_ref, w1_ref, b1_ref, wsc_ref, bsc_ref,
                w2_ref, b2_ref, out_ref):
    p = pl.program_id(0)
    o1 = _out1_of(nbr_ref[...], ce_ref[...], w1_ref, b1_ref)
    u = lax.dot_general(o1, wsc_ref[...], (((1,), (1,)), ((), ())),
                        preferred_element_type=jnp.float32) + bsc_ref[0:1]
    v = lax.dot_general(o1, w2_ref[...], (((1,), (1,)), ((), ())),
                        preferred_element_type=jnp.float32) + b2_ref[0:1]
    o2 = u + _swish_k(v)

    @pl.when(p == 0)
    def _():
        out_ref[...] = o2

    @pl.when(p != 0)
    def _():
        out_ref[...] = jnp.maximum(out_ref[...], o2)


def _final(nbr, ce, w1p, b1t, wscp, bsct, w2p, b2t):
    m = nbr.shape[0]
    return pl.pallas_call(
        _final_body,
        grid=(m // _BLK,),
        in_specs=[pl.BlockSpec((_BLK, 32), lambda p: (p, 0)),
                  pl.BlockSpec((_BLK, 32), lambda p: (p, 0)),
                  pl.BlockSpec((64, 64), lambda p: (0, 0)),
                  pl.BlockSpec((8, 64), lambda p: (0, 0)),
                  pl.BlockSpec((128, 64), lambda p: (0, 0)),
                  pl.BlockSpec((8, 128), lambda p: (0, 0)),
                  pl.BlockSpec((128, 64), lambda p: (0, 0)),
                  pl.BlockSpec((8, 128), lambda p: (0, 0))],
        out_specs=pl.BlockSpec((_BLK, 128), lambda p: (0, 0)),
        out_shape=jax.ShapeDtypeStruct((_BLK, 128), jnp.float32),
    )(nbr, ce, w1p, b1t, wscp, bsct, w2p, b2t)


def _bn_fold(w, bias, gamma, beta, mean_x, cov_x, eps=1e-5):
    mu = w @ mean_x + bias
    var = jnp.sum((w @ cov_x) * w, axis=1)
    s = gamma / jnp.sqrt(jnp.maximum(var, 0.0) + eps)
    return w * s[:, None], s * (bias - mu) + beta


def _tile8(b):
    return jnp.tile(b[None, :], (8, 1))


def kernel(xyz, points, rb1_w, rb1_b, rb1_g, rb1_beta, rb2_w, rb2_b, rb2_g,
           rb2_beta, rb2_sc_w, rb2_sc_b, rb2_sc_g, rb2_sc_beta):
    B, N, _ = xyz.shape
    D = points.shape[-1]
    S, K = NPOINT, NSAMPLE
    M = B * S * K

    xr = xyz[..., 0].reshape(B, N // 128, 128)
    yr = xyz[..., 1].reshape(B, N // 128, 128)
    zr = xyz[..., 2].reshape(B, N // 128, 128)
    fps_idx, nx, ny, nz = _fps(xr, yr, zr)
    new_xyz = jnp.stack([nx, ny, nz], axis=-1)

    p2 = (xyz[..., 0] * xyz[..., 0] + xyz[..., 1] * xyz[..., 1]
          + xyz[..., 2] * xyz[..., 2])
    p8 = jnp.concatenate([xyz, p2[..., None],
                          jnp.zeros((B, N, 4), jnp.float32)], axis=-1)
    qs = jnp.concatenate([nx[:, None], ny[:, None], nz[:, None],
                          jnp.zeros((B, 5, S), jnp.float32)], axis=1)
    idxg = _knn(p8, qs)

    nbr_idx = jnp.transpose(idxg, (2, 0, 1)).reshape(M)
    fps_glob = fps_idx + jnp.arange(B, dtype=jnp.int32)[:, None] * N
    ctr_idx = jnp.tile(fps_glob.reshape(-1), K)
    idx_all = jnp.concatenate([nbr_idx, ctr_idx]).reshape(-1, 128)
    g = _gather_sc(points.reshape(B * N, D), idx_all)
    nbr = g[:M]
    cexp = g[M:]

    gx, sx = _stats_x(nbr, cexp)
    mean_x = sx[0] / M
    cov_x = gx / M - jnp.outer(mean_x, mean_x)
    w1p, b1p = _bn_fold(rb1_w, rb1_b, rb1_g, rb1_beta, mean_x, cov_x)

    g1, s1 = _stats_o1(nbr, cexp, w1p, _tile8(b1p))
    mean1 = s1[0] / M
    cov1 = g1 / M - jnp.outer(mean1, mean1)
    w2p, b2p = _bn_fold(rb2_w, rb2_b, rb2_g, rb2_beta, mean1, cov1)
    wscp, bscp = _bn_fold(rb2_sc_w, rb2_sc_b, rb2_sc_g, rb2_sc_beta,
                          mean1, cov1)

    outf = _final(nbr, cexp, w1p, _tile8(b1p), wscp, _tile8(bscp),
                  w2p, _tile8(b2p))
    out = outf.reshape(B, S, 128).transpose(0, 2, 1)
    return (new_xyz, out)


# speedup vs baseline: 1.0942x; 1.0942x over previous
import functools

import jax
import jax.numpy as jnp
from jax import lax
from jax.experimental import pallas as pl
from jax.experimental.pallas import tpu as pltpu
from jax.experimental.pallas import tpu_sc as plsc

NPOINT = 512
NSAMPLE = 32


def _fps_body(xr_ref, yr_ref, zr_ref, fps_ref, nx_ref, ny_ref, nz_ref,
              dmin_ref):
    X = xr_ref[...]
    Y = yr_ref[...]
    Z = zr_ref[...]
    B = X.shape[0]
    pos = (lax.broadcasted_iota(jnp.int32, X.shape, 1) * 128
           + lax.broadcasted_iota(jnp.int32, X.shape, 2))
    lane = lax.broadcasted_iota(jnp.int32, (B, NPOINT), 1)
    BIGI = jnp.int32(1 << 30)

    def coords_at(sel):
        m = pos == sel[:, None, None]
        cx = jnp.sum(jnp.where(m, X, 0.0), axis=(1, 2))
        cy = jnp.sum(jnp.where(m, Y, 0.0), axis=(1, 2))
        cz = jnp.sum(jnp.where(m, Z, 0.0), axis=(1, 2))
        return cx, cy, cz

    cx, cy, cz = coords_at(jnp.zeros((B,), jnp.int32))
    fps_ref[...] = jnp.zeros((B, NPOINT), jnp.int32)
    nx_ref[...] = jnp.where(lane == 0, cx[:, None], 0.0)
    ny_ref[...] = jnp.where(lane == 0, cy[:, None], 0.0)
    nz_ref[...] = jnp.where(lane == 0, cz[:, None], 0.0)
    dmin_ref[...] = jnp.full(X.shape, jnp.inf, jnp.float32)

    def body(i, carry):
        cx, cy, cz = carry
        d = ((X - cx[:, None, None]) ** 2 + (Y - cy[:, None, None]) ** 2
             + (Z - cz[:, None, None]) ** 2)
        dmin = jnp.minimum(dmin_ref[...], d)
        dmin_ref[...] = dmin
        m = jnp.max(dmin, axis=(1, 2))
        sel = jnp.min(jnp.where(dmin == m[:, None, None], pos, BIGI),
                      axis=(1, 2))
        fps_ref[...] = jnp.where(lane == i, sel[:, None], fps_ref[...])
        cx, cy, cz = coords_at(sel)
        nx_ref[...] = jnp.where(lane == i, cx[:, None], nx_ref[...])
        ny_ref[...] = jnp.where(lane == i, cy[:, None], ny_ref[...])
        nz_ref[...] = jnp.where(lane == i, cz[:, None], nz_ref[...])
        return cx, cy, cz

    lax.fori_loop(1, NPOINT, body, (cx, cy, cz))


def _fps(xr, yr, zr):
    B = xr.shape[0]
    return pl.pallas_call(
        _fps_body,
        out_shape=[jax.ShapeDtypeStruct((B, NPOINT), jnp.int32),
                   jax.ShapeDtypeStruct((B, NPOINT), jnp.float32),
                   jax.ShapeDtypeStruct((B, NPOINT), jnp.float32),
                   jax.ShapeDtypeStruct((B, NPOINT), jnp.float32)],
        scratch_shapes=[pltpu.VMEM(xr.shape, jnp.float32)],
    )(xr, yr, zr)


_SB = 128


def _knn_body(p8_ref, qs_ref, idx_ref, d_ref):
    b = pl.program_id(0)
    N = p8_ref.shape[1]
    P = p8_ref[0]
    QT = qs_ref[0]
    qx = QT[0:1]
    qy = QT[1:2]
    qz = QT[2:3]
    q2 = qx * qx + qy * qy + qz * qz
    E = lax.dot_general(P.astype(jnp.bfloat16), QT.astype(jnp.bfloat16),
                        (((1,), (0,)), ((), ())),
                        preferred_element_type=jnp.float32)
    p2col = P[:, 3:4]
    d_ref[...] = (-2.0 * E + q2) + p2col

    BIGI = jnp.int32(1 << 30)

    def body(k, _):
        Dv = d_ref[...]
        m = jnp.min(Dv, axis=0, keepdims=True)
        rid = lax.broadcasted_iota(jnp.int32, Dv.shape, 0)
        a = jnp.min(jnp.where(Dv == m, rid, BIGI), axis=0,
                    keepdims=True)
        d_ref[...] = jnp.where(rid == a, jnp.inf, Dv)
        kio = lax.broadcasted_iota(jnp.int32, idx_ref.shape, 2)
        idx_ref[...] = jnp.where(kio == k, (a + b * N)[:, :, None],
                                 idx_ref[...])
        return 0

    lax.fori_loop(0, NSAMPLE, body, 0)


def _knn(p8, qs):
    B, N, _ = p8.shape
    return pl.pallas_call(
        _knn_body,
        grid=(B, NPOINT // _SB),
        in_specs=[
            pl.BlockSpec((1, N, 8), lambda b, s: (b, 0, 0)),
            pl.BlockSpec((1, 8, _SB), lambda b, s: (b, 0, s)),
        ],
        out_specs=pl.BlockSpec((1, _SB, NSAMPLE), lambda b, s: (b, s, 0)),
        out_shape=jax.ShapeDtypeStruct((B, NPOINT, NSAMPLE), jnp.int32),
        scratch_shapes=[pltpu.VMEM((N, _SB), jnp.float32)],
    )(p8, qs)


_NC, _NS = 2, 16
_NW = _NC * _NS
_CH = 128


def _gather_sc(pts, idx2d):
    n_rows = idx2d.shape[0] * idx2d.shape[1]
    rows_w = n_rows // _NW
    nch = rows_w // _CH
    d = pts.shape[1]
    mesh = plsc.VectorSubcoreMesh(core_axis_name="c", subcore_axis_name="s")

    @functools.partial(
        pl.kernel,
        out_type=jax.ShapeDtypeStruct((n_rows, d), jnp.float32),
        mesh=mesh,
        compiler_params=pltpu.CompilerParams(use_tc_tiling_on_sc=False),
        scratch_types=[
            pltpu.VMEM((nch, _CH), jnp.int32),
            pltpu.VMEM((_CH, d), jnp.float32),
            pltpu.VMEM((_CH, d), jnp.float32),
            pltpu.SemaphoreType.DMA,
            pltpu.SemaphoreType.DMA,
        ],
    )
    def k(pts_hbm, idx_hbm, out_hbm, idx_v, buf0, buf1, sem0, sem1):
        wid = lax.axis_index("s") * _NC + lax.axis_index("c")
        pltpu.sync_copy(idx_hbm.at[pl.ds(wid * nch, nch)], idx_v)
        obase = wid * rows_w

        def pair(jj, carry):
            j0 = 2 * jj
            c0 = pltpu.async_copy(pts_hbm.at[idx_v.at[j0]], buf0, sem0)
            c1 = pltpu.async_copy(pts_hbm.at[idx_v.at[j0 + 1]], buf1, sem1)
            c0.wait()
            pltpu.sync_copy(buf0, out_hbm.at[pl.ds(obase + j0 * _CH, _CH)])
            c1.wait()
            pltpu.sync_copy(buf1,
                            out_hbm.at[pl.ds(obase + (j0 + 1) * _CH, _CH)])
            return carry

        lax.fori_loop(0, nch // 2, pair, 0)

    return k(pts, idx2d)


_BLK = 2048


def _stats_x_body(nbr_ref, ce_ref, g_ref, s_ref):
    p = pl.program_id(0)
    nb = nbr_ref[...]
    ce = ce_ref[...]
    x = jnp.concatenate([nb - ce, ce], axis=1)
    g = lax.dot_general(x, x, (((0,), (0,)), ((), ())),
                        preferred_element_type=jnp.float32,
                        precision=lax.Precision.HIGHEST)
    ones = jnp.ones((8, _BLK), jnp.float32)
    s = lax.dot_general(ones, x, (((1,), (0,)), ((), ())),
                        preferred_element_type=jnp.float32,
                        precision=lax.Precision.HIGHEST)

    @pl.when(p == 0)
    def _():
        g_ref[...] = jnp.zeros_like(g_ref)
        s_ref[...] = jnp.zeros_like(s_ref)

    g_ref[...] += g
    s_ref[...] += s


def _stats_x(nbr, ce):
    m = nbr.shape[0]
    return pl.pallas_call(
        _stats_x_body,
        grid=(m // _BLK,),
        in_specs=[pl.BlockSpec((_BLK, 32), lambda p: (p, 0)),
                  pl.BlockSpec((_BLK, 32), lambda p: (p, 0))],
        out_specs=[pl.BlockSpec((64, 64), lambda p: (0, 0)),
                   pl.BlockSpec((8, 64), lambda p: (0, 0))],
        out_shape=[jax.ShapeDtypeStruct((64, 64), jnp.float32),
                   jax.ShapeDtypeStruct((8, 64), jnp.float32)],
    )(nbr, ce)


def _swish_k(t):
    return t * (1.0 / (1.0 + jnp.exp(-t)))


def _out1_of(nb, ce, w1_ref, b1_ref):
    x = jnp.concatenate([nb - ce, ce], axis=1)
    t = lax.dot_general(x, w1_ref[...], (((1,), (1,)), ((), ())),
                        preferred_element_type=jnp.float32) + b1_ref[0:1]
    return x + _swish_k(t)


def _stats_o1_body(nbr_ref, ce_ref, w1_ref, b1_ref, g_ref, s_ref):
    p = pl.program_id(0)
    o1 = _out1_of(nbr_ref[...], ce_ref[...], w1_ref, b1_ref)
    g = lax.dot_general(o1, o1, (((0,), (0,)), ((), ())),
                        preferred_element_type=jnp.float32,
                        precision=lax.Precision.HIGHEST)
    ones = jnp.ones((8, _BLK), jnp.float32)
    s = lax.dot_general(ones, o1, (((1,), (0,)), ((), ())),
                        preferred_element_type=jnp.float32,
                        precision=lax.Precision.HIGHEST)

    @pl.when(p == 0)
    def _():
        g_ref[...] = jnp.zeros_like(g_ref)
        s_ref[...] = jnp.zeros_like(s_ref)

    g_ref[...] += g
    s_ref[...] += s


def _stats_o1(nbr, ce, w1p, b1t):
    m = nbr.shape[0]
    return pl.pallas_call(
        _stats_o1_body,
        grid=(m // _BLK,),
        in_specs=[pl.BlockSpec((_BLK, 32), lambda p: (p, 0)),
                  pl.BlockSpec((_BLK, 32), lambda p: (p, 0)),
                  pl.BlockSpec((64, 64), lambda p: (0, 0)),
                  pl.BlockSpec((8, 64), lambda p: (0, 0))],
        out_specs=[pl.BlockSpec((64, 64), lambda p: (0, 0)),
                   pl.BlockSpec((8, 64), lambda p: (0, 0))],
        out_shape=[jax.ShapeDtypeStruct((64, 64), jnp.float32),
                   jax.ShapeDtypeStruct((8, 64), jnp.float32)],
    )(nbr, ce, w1p, b1t)


def _final_body(nbr_ref, ce_ref, w1_ref, b1_ref, wsc_ref, bsc_ref,
                w2_ref, b2_ref, out_ref):
    p = pl.program_id(0)
    o1 = _out1_of(nbr_ref[...], ce_ref[...], w1_ref, b1_ref)
    u = lax.dot_general(o1, wsc_ref[...], (((1,), (1,)), ((), ())),
                        preferred_element_type=jnp.float32) + bsc_ref[0:1]
    v = lax.dot_general(o1, w2_ref[...], (((1,), (1,)), ((), ())),
                        preferred_element_type=jnp.float32) + b2_ref[0:1]
    o2 = u + _swish_k(v)

    @pl.when(p == 0)
    def _():
        out_ref[...] = o2

    @pl.when(p != 0)
    def _():
        out_ref[...] = jnp.maximum(out_ref[...], o2)


def _final(nbr, ce, w1p, b1t, wscp, bsct, w2p, b2t):
    m = nbr.shape[0]
    return pl.pallas_call(
        _final_body,
        grid=(m // _BLK,),
        in_specs=[pl.BlockSpec((_BLK, 32), lambda p: (p, 0)),
                  pl.BlockSpec((_BLK, 32), lambda p: (p, 0)),
                  pl.BlockSpec((64, 64), lambda p: (0, 0)),
                  pl.BlockSpec((8, 64), lambda p: (0, 0)),
                  pl.BlockSpec((128, 64), lambda p: (0, 0)),
                  pl.BlockSpec((8, 128), lambda p: (0, 0)),
                  pl.BlockSpec((128, 64), lambda p: (0, 0)),
                  pl.BlockSpec((8, 128), lambda p: (0, 0))],
        out_specs=pl.BlockSpec((_BLK, 128), lambda p: (0, 0)),
        out_shape=jax.ShapeDtypeStruct((_BLK, 128), jnp.float32),
    )(nbr, ce, w1p, b1t, wscp, bsct, w2p, b2t)


def _bn_fold(w, bias, gamma, beta, mean_x, cov_x, eps=1e-5):
    mu = w @ mean_x + bias
    var = jnp.sum((w @ cov_x) * w, axis=1)
    s = gamma / jnp.sqrt(jnp.maximum(var, 0.0) + eps)
    return w * s[:, None], s * (bias - mu) + beta


def _tile8(b):
    return jnp.tile(b[None, :], (8, 1))


def kernel(xyz, points, rb1_w, rb1_b, rb1_g, rb1_beta, rb2_w, rb2_b, rb2_g,
           rb2_beta, rb2_sc_w, rb2_sc_b, rb2_sc_g, rb2_sc_beta):
    B, N, _ = xyz.shape
    D = points.shape[-1]
    S, K = NPOINT, NSAMPLE
    M = B * S * K

    xr = xyz[..., 0].reshape(B, N // 128, 128)
    yr = xyz[..., 1].reshape(B, N // 128, 128)
    zr = xyz[..., 2].reshape(B, N // 128, 128)
    fps_idx, nx, ny, nz = _fps(xr, yr, zr)
    new_xyz = jnp.stack([nx, ny, nz], axis=-1)

    p2 = (xyz[..., 0] * xyz[..., 0] + xyz[..., 1] * xyz[..., 1]
          + xyz[..., 2] * xyz[..., 2])
    p8 = jnp.concatenate([xyz, p2[..., None],
                          jnp.zeros((B, N, 4), jnp.float32)], axis=-1)
    qs = jnp.concatenate([nx[:, None], ny[:, None], nz[:, None],
                          jnp.zeros((B, 5, S), jnp.float32)], axis=1)
    idxg = _knn(p8, qs)

    nbr_idx = jnp.transpose(idxg, (2, 0, 1)).reshape(M)
    fps_glob = fps_idx + jnp.arange(B, dtype=jnp.int32)[:, None] * N
    ctr_idx = jnp.tile(fps_glob.reshape(-1), K)
    idx_all = jnp.concatenate([nbr_idx, ctr_idx]).reshape(-1, 128)
    g = _gather_sc(points.reshape(B * N, D), idx_all)
    nbr = g[:M]
    cexp = g[M:]

    gx, sx = _stats_x(nbr, cexp)
    mean_x = sx[0] / M
    cov_x = gx / M - jnp.outer(mean_x, mean_x)
    w1p, b1p = _bn_fold(rb1_w, rb1_b, rb1_g, rb1_beta, mean_x, cov_x)

    g1, s1 = _stats_o1(nbr, cexp, w1p, _tile8(b1p))
    mean1 = s1[0] / M
    cov1 = g1 / M - jnp.outer(mean1, mean1)
    w2p, b2p = _bn_fold(rb2_w, rb2_b, rb2_g, rb2_beta, mean1, cov1)
    wscp, bscp = _bn_fold(rb2_sc_w, rb2_sc_b, rb2_sc_g, rb2_sc_beta,
                          mean1, cov1)

    outf = _final(nbr, cexp, w1p, _tile8(b1p), wscp, _tile8(bscp),
                  w2p, _tile8(b2p))
    out = outf.reshape(B, S, 128).transpose(0, 2, 1)
    return (new_xyz, out)


# speedup vs baseline: 1.4081x; 1.2869x over previous
import functools

import jax
import jax.numpy as jnp
from jax import lax
from jax.experimental import pallas as pl
from jax.experimental.pallas import tpu as pltpu
from jax.experimental.pallas import tpu_sc as plsc

NPOINT = 512
NSAMPLE = 32


def _fps_body(xr_ref, yr_ref, zr_ref, fps_ref, nx_ref, ny_ref, nz_ref,
              dmin_ref):
    X = xr_ref[...]
    Y = yr_ref[...]
    Z = zr_ref[...]
    B = X.shape[0]
    pos = (lax.broadcasted_iota(jnp.int32, X.shape, 1) * 128
           + lax.broadcasted_iota(jnp.int32, X.shape, 2))
    lane = lax.broadcasted_iota(jnp.int32, (B, NPOINT), 1)
    BIGI = jnp.int32(1 << 30)

    def coords_at(sel):
        m = pos == sel[:, None, None]
        cx = jnp.sum(jnp.where(m, X, 0.0), axis=(1, 2))
        cy = jnp.sum(jnp.where(m, Y, 0.0), axis=(1, 2))
        cz = jnp.sum(jnp.where(m, Z, 0.0), axis=(1, 2))
        return cx, cy, cz

    cx, cy, cz = coords_at(jnp.zeros((B,), jnp.int32))
    fps_ref[...] = jnp.zeros((B, NPOINT), jnp.int32)
    nx_ref[...] = jnp.where(lane == 0, cx[:, None], 0.0)
    ny_ref[...] = jnp.where(lane == 0, cy[:, None], 0.0)
    nz_ref[...] = jnp.where(lane == 0, cz[:, None], 0.0)
    dmin_ref[...] = jnp.full(X.shape, jnp.inf, jnp.float32)

    def body(i, carry):
        cx, cy, cz = carry
        d = ((X - cx[:, None, None]) ** 2 + (Y - cy[:, None, None]) ** 2
             + (Z - cz[:, None, None]) ** 2)
        dmin = jnp.minimum(dmin_ref[...], d)
        dmin_ref[...] = dmin
        m = jnp.max(dmin, axis=(1, 2))
        sel = jnp.min(jnp.where(dmin == m[:, None, None], pos, BIGI),
                      axis=(1, 2))
        fps_ref[...] = jnp.where(lane == i, sel[:, None], fps_ref[...])
        cx, cy, cz = coords_at(sel)
        nx_ref[...] = jnp.where(lane == i, cx[:, None], nx_ref[...])
        ny_ref[...] = jnp.where(lane == i, cy[:, None], ny_ref[...])
        nz_ref[...] = jnp.where(lane == i, cz[:, None], nz_ref[...])
        return cx, cy, cz

    lax.fori_loop(1, NPOINT, body, (cx, cy, cz))


def _fps(xr, yr, zr):
    B = xr.shape[0]
    return pl.pallas_call(
        _fps_body,
        out_shape=[jax.ShapeDtypeStruct((B, NPOINT), jnp.int32),
                   jax.ShapeDtypeStruct((B, NPOINT), jnp.float32),
                   jax.ShapeDtypeStruct((B, NPOINT), jnp.float32),
                   jax.ShapeDtypeStruct((B, NPOINT), jnp.float32)],
        scratch_shapes=[pltpu.VMEM(xr.shape, jnp.float32)],
    )(xr, yr, zr)


_SB = 256


def _knn_body(p8_ref, qs_ref, idx_ref, d_ref):
    b = pl.program_id(0)
    N = p8_ref.shape[1]
    P = p8_ref[0]
    QT = qs_ref[0]
    qx = QT[0:1]
    qy = QT[1:2]
    qz = QT[2:3]
    q2 = qx * qx + qy * qy + qz * qz
    E = lax.dot_general(P.astype(jnp.bfloat16), QT.astype(jnp.bfloat16),
                        (((1,), (0,)), ((), ())),
                        preferred_element_type=jnp.float32)
    p2col = P[:, 3:4]
    d_ref[...] = (-2.0 * E + q2) + p2col

    BIGI = jnp.int32(1 << 30)

    def body(k, _):
        Dv = d_ref[...]
        m = jnp.min(Dv, axis=0, keepdims=True)
        rid = lax.broadcasted_iota(jnp.int32, Dv.shape, 0)
        a = jnp.min(jnp.where(Dv == m, rid, BIGI), axis=0,
                    keepdims=True)
        d_ref[...] = jnp.where(rid == a, jnp.inf, Dv)
        kio = lax.broadcasted_iota(jnp.int32, idx_ref.shape, 2)
        idx_ref[...] = jnp.where(kio == k, (a + b * N)[:, :, None],
                                 idx_ref[...])
        return 0

    lax.fori_loop(0, NSAMPLE, body, 0)


def _knn(p8, qs):
    B, N, _ = p8.shape
    return pl.pallas_call(
        _knn_body,
        grid=(B, NPOINT // _SB),
        in_specs=[
            pl.BlockSpec((1, N, 8), lambda b, s: (b, 0, 0)),
            pl.BlockSpec((1, 8, _SB), lambda b, s: (b, 0, s)),
        ],
        out_specs=pl.BlockSpec((1, _SB, NSAMPLE), lambda b, s: (b, s, 0)),
        out_shape=jax.ShapeDtypeStruct((B, NPOINT, NSAMPLE), jnp.int32),
        scratch_shapes=[pltpu.VMEM((N, _SB), jnp.float32)],
    )(p8, qs)


_NC, _NS = 2, 16
_NW = _NC * _NS
_CH = 128


def _gather_sc(pts, idx2d):
    n_rows = idx2d.shape[0] * idx2d.shape[1]
    rows_w = n_rows // _NW
    nch = rows_w // _CH
    d = pts.shape[1]
    mesh = plsc.VectorSubcoreMesh(core_axis_name="c", subcore_axis_name="s")

    @functools.partial(
        pl.kernel,
        out_type=jax.ShapeDtypeStruct((n_rows, d), jnp.float32),
        mesh=mesh,
        compiler_params=pltpu.CompilerParams(use_tc_tiling_on_sc=False),
        scratch_types=[
            pltpu.VMEM((nch, _CH), jnp.int32),
            pltpu.VMEM((_CH, d), jnp.float32),
            pltpu.VMEM((_CH, d), jnp.float32),
            pltpu.SemaphoreType.DMA,
            pltpu.SemaphoreType.DMA,
        ],
    )
    def k(pts_hbm, idx_hbm, out_hbm, idx_v, buf0, buf1, sem0, sem1):
        wid = lax.axis_index("s") * _NC + lax.axis_index("c")
        pltpu.sync_copy(idx_hbm.at[pl.ds(wid * nch, nch)], idx_v)
        obase = wid * rows_w

        def pair(jj, carry):
            j0 = 2 * jj
            c0 = pltpu.async_copy(pts_hbm.at[idx_v.at[j0]], buf0, sem0)
            c1 = pltpu.async_copy(pts_hbm.at[idx_v.at[j0 + 1]], buf1, sem1)
            c0.wait()
            pltpu.sync_copy(buf0, out_hbm.at[pl.ds(obase + j0 * _CH, _CH)])
            c1.wait()
            pltpu.sync_copy(buf1,
                            out_hbm.at[pl.ds(obase + (j0 + 1) * _CH, _CH)])
            return carry

        lax.fori_loop(0, nch // 2, pair, 0)

    return k(pts, idx2d)


_BLK = 2048


def _stats_x_body(nbr_ref, ce_ref, g_ref, s_ref):
    p = pl.program_id(0)
    nb = nbr_ref[...]
    ce = ce_ref[...]
    x = jnp.concatenate([nb - ce, ce], axis=1)
    g = lax.dot_general(x, x, (((0,), (0,)), ((), ())),
                        preferred_element_type=jnp.float32,
                        precision=lax.Precision.HIGHEST)
    ones = jnp.ones((8, _BLK), jnp.float32)
    s = lax.dot_general(ones, x, (((1,), (0,)), ((), ())),
                        preferred_element_type=jnp.float32,
                        precision=lax.Precision.HIGHEST)

    @pl.when(p == 0)
    def _():
        g_ref[...] = jnp.zeros_like(g_ref)
        s_ref[...] = jnp.zeros_like(s_ref)

    g_ref[...] += g
    s_ref[...] += s


def _stats_x(nbr, ce):
    m = nbr.shape[0]
    return pl.pallas_call(
        _stats_x_body,
        grid=(m // _BLK,),
        in_specs=[pl.BlockSpec((_BLK, 32), lambda p: (p, 0)),
                  pl.BlockSpec((_BLK, 32), lambda p: (p, 0))],
        out_specs=[pl.BlockSpec((64, 64), lambda p: (0, 0)),
                   pl.BlockSpec((8, 64), lambda p: (0, 0))],
        out_shape=[jax.ShapeDtypeStruct((64, 64), jnp.float32),
                   jax.ShapeDtypeStruct((8, 64), jnp.float32)],
    )(nbr, ce)


def _swish_k(t):
    return t * (1.0 / (1.0 + jnp.exp(-t)))


def _out1_of(nb, ce, w1_ref, b1_ref):
    x = jnp.concatenate([nb - ce, ce], axis=1)
    t = lax.dot_general(x, w1_ref[...], (((1,), (1,)), ((), ())),
                        preferred_element_type=jnp.float32) + b1_ref[0:1]
    return x + _swish_k(t)


def _stats_o1_body(nbr_ref, ce_ref, w1_ref, b1_ref, g_ref, s_ref):
    p = pl.program_id(0)
    o1 = _out1_of(nbr_ref[...], ce_ref[...], w1_ref, b1_ref)
    g = lax.dot_general(o1, o1, (((0,), (0,)), ((), ())),
                        preferred_element_type=jnp.float32,
                        precision=lax.Precision.HIGHEST)
    ones = jnp.ones((8, _BLK), jnp.float32)
    s = lax.dot_general(ones, o1, (((1,), (0,)), ((), ())),
                        preferred_element_type=jnp.float32,
                        precision=lax.Precision.HIGHEST)

    @pl.when(p == 0)
    def _():
        g_ref[...] = jnp.zeros_like(g_ref)
        s_ref[...] = jnp.zeros_like(s_ref)

    g_ref[...] += g
    s_ref[...] += s


def _stats_o1(nbr, ce, w1p, b1t):
    m = nbr.shape[0]
    return pl.pallas_call(
        _stats_o1_body,
        grid=(m // _BLK,),
        in_specs=[pl.BlockSpec((_BLK, 32), lambda p: (p, 0)),
                  pl.BlockSpec((_BLK, 32), lambda p: (p, 0)),
                  pl.BlockSpec((64, 64), lambda p: (0, 0)),
                  pl.BlockSpec((8, 64), lambda p: (0, 0))],
        out_specs=[pl.BlockSpec((64, 64), lambda p: (0, 0)),
                   pl.BlockSpec((8, 64), lambda p: (0, 0))],
        out_shape=[jax.ShapeDtypeStruct((64, 64), jnp.float32),
                   jax.ShapeDtypeStruct((8, 64), jnp.float32)],
    )(nbr, ce, w1p, b1t)


def _final_body(nbr_ref, ce_ref, w1_ref, b1_ref, wsc_ref, bsc_ref,
                w2_ref, b2_ref, out_ref):
    p = pl.program_id(0)
    o1 = _out1_of(nbr_ref[...], ce_ref[...], w1_ref, b1_ref)
    u = lax.dot_general(o1, wsc_ref[...], (((1,), (1,)), ((), ())),
                        preferred_element_type=jnp.float32) + bsc_ref[0:1]
    v = lax.dot_general(o1, w2_ref[...], (((1,), (1,)), ((), ())),
                        preferred_element_type=jnp.float32) + b2_ref[0:1]
    o2 = u + _swish_k(v)

    @pl.when(p == 0)
    def _():
        out_ref[...] = o2

    @pl.when(p != 0)
    def _():
        out_ref[...] = jnp.maximum(out_ref[...], o2)


def _final(nbr, ce, w1p, b1t, wscp, bsct, w2p, b2t):
    m = nbr.shape[0]
    return pl.pallas_call(
        _final_body,
        grid=(m // _BLK,),
        in_specs=[pl.BlockSpec((_BLK, 32), lambda p: (p, 0)),
                  pl.BlockSpec((_BLK, 32), lambda p: (p, 0)),
                  pl.BlockSpec((64, 64), lambda p: (0, 0)),
                  pl.BlockSpec((8, 64), lambda p: (0, 0)),
                  pl.BlockSpec((128, 64), lambda p: (0, 0)),
                  pl.BlockSpec((8, 128), lambda p: (0, 0)),
                  pl.BlockSpec((128, 64), lambda p: (0, 0)),
                  pl.BlockSpec((8, 128), lambda p: (0, 0))],
        out_specs=pl.BlockSpec((_BLK, 128), lambda p: (0, 0)),
        out_shape=jax.ShapeDtypeStruct((_BLK, 128), jnp.float32),
    )(nbr, ce, w1p, b1t, wscp, bsct, w2p, b2t)


def _bn_fold(w, bias, gamma, beta, mean_x, cov_x, eps=1e-5):
    mu = w @ mean_x + bias
    var = jnp.sum((w @ cov_x) * w, axis=1)
    s = gamma / jnp.sqrt(jnp.maximum(var, 0.0) + eps)
    return w * s[:, None], s * (bias - mu) + beta


def _tile8(b):
    return jnp.tile(b[None, :], (8, 1))


def kernel(xyz, points, rb1_w, rb1_b, rb1_g, rb1_beta, rb2_w, rb2_b, rb2_g,
           rb2_beta, rb2_sc_w, rb2_sc_b, rb2_sc_g, rb2_sc_beta):
    B, N, _ = xyz.shape
    D = points.shape[-1]
    S, K = NPOINT, NSAMPLE
    M = B * S * K

    xr = xyz[..., 0].reshape(B, N // 128, 128)
    yr = xyz[..., 1].reshape(B, N // 128, 128)
    zr = xyz[..., 2].reshape(B, N // 128, 128)
    fps_idx, nx, ny, nz = _fps(xr, yr, zr)
    new_xyz = jnp.stack([nx, ny, nz], axis=-1)

    p2 = (xyz[..., 0] * xyz[..., 0] + xyz[..., 1] * xyz[..., 1]
          + xyz[..., 2] * xyz[..., 2])
    p8 = jnp.concatenate([xyz, p2[..., None],
                          jnp.zeros((B, N, 4), jnp.float32)], axis=-1)
    qs = jnp.concatenate([nx[:, None], ny[:, None], nz[:, None],
                          jnp.zeros((B, 5, S), jnp.float32)], axis=1)
    idxg = _knn(p8, qs)

    nbr_idx = jnp.transpose(idxg, (2, 0, 1)).reshape(M)
    fps_glob = fps_idx + jnp.arange(B, dtype=jnp.int32)[:, None] * N
    ctr_idx = jnp.tile(fps_glob.reshape(-1), K)
    idx_all = jnp.concatenate([nbr_idx, ctr_idx]).reshape(-1, 128)
    g = _gather_sc(points.reshape(B * N, D), idx_all)
    nbr = g[:M]
    cexp = g[M:]

    gx, sx = _stats_x(nbr, cexp)
    mean_x = sx[0] / M
    cov_x = gx / M - jnp.outer(mean_x, mean_x)
    w1p, b1p = _bn_fold(rb1_w, rb1_b, rb1_g, rb1_beta, mean_x, cov_x)

    g1, s1 = _stats_o1(nbr, cexp, w1p, _tile8(b1p))
    mean1 = s1[0] / M
    cov1 = g1 / M - jnp.outer(mean1, mean1)
    w2p, b2p = _bn_fold(rb2_w, rb2_b, rb2_g, rb2_beta, mean1, cov1)
    wscp, bscp = _bn_fold(rb2_sc_w, rb2_sc_b, rb2_sc_g, rb2_sc_beta,
                          mean1, cov1)

    outf = _final(nbr, cexp, w1p, _tile8(b1p), wscp, _tile8(bscp),
                  w2p, _tile8(b2p))
    out = outf.reshape(B, S, 128).transpose(0, 2, 1)
    return (new_xyz, out)
